# baseline (speedup 1.0000x reference)
import functools

import jax
import jax.numpy as jnp
from jax import lax
from jax.experimental import pallas as pl
from jax.experimental.pallas import tpu as pltpu
from jax.experimental.pallas import tpu_sc as plsc

_CH = 64


def _combine_weights(xb, wg, bg):
    logits = jnp.dot(xb, wg, preferred_element_type=jnp.float32) + bg
    m = jnp.max(logits, axis=-1, keepdims=True)
    p = jnp.exp(logits - m)
    p = p / jnp.sum(p, axis=-1, keepdims=True)

    i1 = jnp.argmax(p, axis=-1)[:, None]
    top1 = jnp.max(p, axis=-1, keepdims=True)
    cols = jax.lax.broadcasted_iota(jnp.int32, p.shape, 1)
    p2 = jnp.where(cols == i1, -jnp.inf, p)
    i2 = jnp.argmax(p2, axis=-1)[:, None]
    top2 = jnp.max(p2, axis=-1, keepdims=True)
    denom = top1 + top2
    c = (jnp.where(cols == i1, top1, 0.0)
         + jnp.where(cols == i2, top2, 0.0)) / denom
    return c, i1, i2


def _gate_block(x_ref, wg_ref, bg_ref, pair_ref):
    _, i1, i2 = _combine_weights(x_ref[...], wg_ref[...], bg_ref[...])
    lo = jnp.minimum(i1, i2)
    hi = jnp.maximum(i1, i2)
    pair_ref[...] = lo * 8 + hi


def _ffn_block(xs_ref, wg_ref, bg_ref, w1_ref, b1_ref, w2_ref, b2_ref, o_ref,
               *, num_experts):
    E = num_experts
    xb = xs_ref[...]
    cs, _, _ = _combine_weights(xb, wg_ref[...], bg_ref[...])
    xb16 = xb.astype(jnp.bfloat16)
    o_ref[...] = jnp.dot(cs, b2_ref[...], preferred_element_type=jnp.float32)
    for e in range(E):
        present = jnp.any(cs[:, e] > 0.0)

        @pl.when(present)
        def _():
            h = jnp.dot(xb16, w1_ref[e], preferred_element_type=jnp.float32)
            h = jnp.maximum(h + b1_ref[e], 0.0) * cs[:, e:e + 1]
            o_ref[...] += jnp.dot(h.astype(jnp.bfloat16), w2_ref[e],
                                  preferred_element_type=jnp.float32)


def _sc_move(rows, perm2, gather):
    N, D = rows.shape
    info = plsc.get_sparse_core_info()
    NW = info.num_cores * info.num_subcores
    rows_pw = N // NW
    T = rows_pw // _CH
    mesh = plsc.VectorSubcoreMesh(core_axis_name="c", subcore_axis_name="s")

    @functools.partial(
        pl.kernel, mesh=mesh,
        out_type=jax.ShapeDtypeStruct((N, D), jnp.float32),
        scratch_types=[pltpu.VMEM((T, _CH), jnp.int32),
                       pltpu.VMEM((_CH, D), jnp.float32),
                       pltpu.VMEM((_CH, D), jnp.float32),
                       pltpu.SemaphoreType.DMA((2,)),
                       pltpu.SemaphoreType.DMA((2,))],
    )
    def k(rows_hbm, perm_hbm, out_hbm, idx_v, buf0, buf1, gsem, wsem):
        wid = lax.axis_index("s") * info.num_cores + lax.axis_index("c")
        base = wid * rows_pw
        bufs = (buf0, buf1)
        pltpu.sync_copy(perm_hbm.at[pl.ds(wid * T, T)], idx_v)

        def start_read(t, b):
            if gather:
                return pltpu.async_copy(rows_hbm.at[idx_v.at[t]], bufs[b],
                                        gsem.at[b])
            return pltpu.async_copy(
                rows_hbm.at[pl.ds(base + t * _CH, _CH)], bufs[b], gsem.at[b])

        def start_write(t, b):
            if gather:
                return pltpu.async_copy(
                    bufs[b], out_hbm.at[pl.ds(base + t * _CH, _CH)], wsem.at[b])
            return pltpu.async_copy(bufs[b], out_hbm.at[idx_v.at[t]],
                                    wsem.at[b])

        reads = [None, None]
        writes = [None, None]
        reads[0] = start_read(0, 0)
        for t in range(T):
            b = t % 2
            nb = (t + 1) % 2
            if t + 1 < T:
                if writes[nb] is not None:
                    writes[nb].wait()
                    writes[nb] = None
                reads[nb] = start_read(t + 1, nb)
            reads[b].wait()
            writes[b] = start_write(t, b)
        for w in writes:
            if w is not None:
                w.wait()

    return k(rows, perm2)


def kernel(x, Wg, bg, W1, b1, W2, b2):
    B, S, H = x.shape
    E, _, F = W1.shape
    N = B * S
    xf = x.reshape(N, H)
    M = 512
    grid = (N // M,)

    pair = pl.pallas_call(
        _gate_block,
        grid=grid,
        in_specs=[
            pl.BlockSpec((M, H), lambda i: (i, 0)),
            pl.BlockSpec((H, E), lambda i: (0, 0)),
            pl.BlockSpec((1, E), lambda i: (0, 0)),
        ],
        out_specs=pl.BlockSpec((M, 1), lambda i: (i, 0)),
        out_shape=jax.ShapeDtypeStruct((N, 1), jnp.int32),
    )(xf, Wg, bg.reshape(1, E))

    perm = jnp.argsort(pair[:, 0]).astype(jnp.int32)
    perm2 = perm.reshape(N // _CH, _CH)

    xs = _sc_move(xf, perm2, gather=True)

    W1a = W1.astype(jnp.bfloat16)
    W2a = W2.astype(jnp.bfloat16)
    ys = pl.pallas_call(
        functools.partial(_ffn_block, num_experts=E),
        grid=grid,
        in_specs=[
            pl.BlockSpec((M, H), lambda i: (i, 0)),
            pl.BlockSpec((H, E), lambda i: (0, 0)),
            pl.BlockSpec((1, E), lambda i: (0, 0)),
            pl.BlockSpec((E, H, F), lambda i: (0, 0, 0)),
            pl.BlockSpec((E, 1, F), lambda i: (0, 0, 0)),
            pl.BlockSpec((E, F, H), lambda i: (0, 0, 0)),
            pl.BlockSpec((E, H), lambda i: (0, 0)),
        ],
        out_specs=pl.BlockSpec((M, H), lambda i: (i, 0)),
        out_shape=jax.ShapeDtypeStruct((N, H), jnp.float32),
    )(xs, Wg, bg.reshape(1, E), W1a, b1.reshape(E, 1, F), W2a, b2)

    out = _sc_move(ys, perm2, gather=False)
    return out.reshape(B, S, H)


# speedup vs baseline: 2.2057x; 2.2057x over previous
import functools

import jax
import jax.numpy as jnp
from jax.experimental import pallas as pl


def _moe_block(x_ref, wg_ref, bg_ref, w1_ref, b1_ref, w2_ref, b2_ref, o_ref,
               *, num_experts):
    xb = x_ref[...]
    logits = jnp.dot(xb, wg_ref[...], preferred_element_type=jnp.float32)
    logits = logits + bg_ref[...]
    m = jnp.max(logits, axis=-1, keepdims=True)
    p = jnp.exp(logits - m)
    p = p / jnp.sum(p, axis=-1, keepdims=True)

    i1 = jnp.argmax(p, axis=-1)[:, None]
    top1 = jnp.max(p, axis=-1, keepdims=True)
    cols = jax.lax.broadcasted_iota(jnp.int32, p.shape, 1)
    p2 = jnp.where(cols == i1, -jnp.inf, p)
    i2 = jnp.argmax(p2, axis=-1)[:, None]
    top2 = jnp.max(p2, axis=-1, keepdims=True)
    denom = top1 + top2

    acc = jnp.zeros_like(xb)
    xb16 = xb.astype(jnp.bfloat16)
    for e in range(num_experts):
        w_e = (jnp.where(i1 == e, top1, 0.0) + jnp.where(i2 == e, top2, 0.0)) / denom
        h = jnp.dot(xb16, w1_ref[e].astype(jnp.bfloat16),
                    preferred_element_type=jnp.float32) + b1_ref[e]
        h = jnp.maximum(h, 0.0)
        y = jnp.dot(h.astype(jnp.bfloat16), w2_ref[e].astype(jnp.bfloat16),
                    preferred_element_type=jnp.float32) + b2_ref[e]
        acc = acc + w_e * y
    o_ref[...] = acc


def kernel(x, Wg, bg, W1, b1, W2, b2):
    B, S, H = x.shape
    E, _, F = W1.shape
    N = B * S
    xf = x.reshape(N, H)
    M = 512
    grid = (N // M,)

    out = pl.pallas_call(
        functools.partial(_moe_block, num_experts=E),
        grid=grid,
        in_specs=[
            pl.BlockSpec((M, H), lambda i: (i, 0)),
            pl.BlockSpec((H, E), lambda i: (0, 0)),
            pl.BlockSpec((1, E), lambda i: (0, 0)),
            pl.BlockSpec((E, H, F), lambda i: (0, 0, 0)),
            pl.BlockSpec((E, 1, F), lambda i: (0, 0, 0)),
            pl.BlockSpec((E, F, H), lambda i: (0, 0, 0)),
            pl.BlockSpec((E, 1, H), lambda i: (0, 0, 0)),
        ],
        out_specs=pl.BlockSpec((M, H), lambda i: (i, 0)),
        out_shape=jax.ShapeDtypeStruct((N, H), jnp.float32),
    )(xf, Wg, bg.reshape(1, E), W1, b1.reshape(E, 1, F), W2, b2.reshape(E, 1, H))
    return out.reshape(B, S, H)
